# Initial kernel scaffold; baseline (speedup 1.0000x reference)
import jax
import jax.numpy as jnp
from jax.experimental import pallas as pl


def kernel(x_cat, tables):
    raise NotImplementedError("write your pallas kernel here")


# speedup vs baseline: 1.7557x; 1.7557x over previous
import functools

import jax
import jax.numpy as jnp
from jax import lax
from jax.experimental import pallas as pl
from jax.experimental.pallas import tpu as pltpu
from jax.experimental.pallas import tpu_sc as plsc

_VOCAB = 100000
_EMB = 16
_LANES = 16
_NW = 32
_IDXW = 128


@functools.lru_cache(maxsize=None)
def _make_gather(n_rows, n_fields):
    nch = 13 if n_fields == 26 else n_fields
    chunk = nch * _IDXW
    per_w = n_rows // _NW
    n_chunks = per_w // chunk
    assert per_w % chunk == 0 and per_w % n_fields == 0

    mesh = plsc.VectorSubcoreMesh(core_axis_name="c", subcore_axis_name="s")

    @functools.partial(
        pl.kernel,
        mesh=mesh,
        compiler_params=pltpu.CompilerParams(use_tc_tiling_on_sc=False),
        out_type=jax.ShapeDtypeStruct((n_rows, _EMB), jnp.float32),
        scratch_types=[
            pltpu.VMEM((chunk,), jnp.int32),
            pltpu.VMEM((chunk, _EMB), jnp.float32),
            pltpu.VMEM((chunk,), jnp.int32),
            pltpu.SemaphoreType.DMA,
        ],
    )
    def gather_kernel(tables_hbm, xcat_hbm, offs_hbm, out_hbm,
                      idx_v, rows_v, offs_v, sem):
        wid = lax.axis_index("s") * 2 + lax.axis_index("c")
        orow0 = wid * per_w
        pltpu.sync_copy(offs_hbm, offs_v)

        def chunk_body(g, carry):
            pltpu.sync_copy(xcat_hbm.at[pl.ds(orow0 + g * chunk, chunk)], idx_v)
            for v in range(chunk // _LANES):
                sl = pl.ds(v * _LANES, _LANES)
                idx_v[sl] = idx_v[sl] + offs_v[sl]
            cps = [
                pltpu.async_copy(
                    tables_hbm.at[idx_v.at[pl.ds(ch * _IDXW, _IDXW)]],
                    rows_v.at[pl.ds(ch * _IDXW, _IDXW)],
                    sem,
                )
                for ch in range(nch)
            ]
            for cp in cps:
                cp.wait()
            pltpu.sync_copy(rows_v, out_hbm.at[pl.ds(orow0 + g * chunk, chunk)])
            return carry

        lax.fori_loop(0, n_chunks, chunk_body, 0)

    return gather_kernel


def kernel(x_cat, tables):
    b, l, f = x_cat.shape
    n_rows = b * l * f
    tables_flat = tables.reshape(f * _VOCAB, _EMB)
    xcat_flat = x_cat.reshape(n_rows)
    nch = 13 if f == 26 else f
    offs = jnp.tile(jnp.arange(f, dtype=jnp.int32) * _VOCAB,
                    (nch * _IDXW) // f)
    out_flat = _make_gather(n_rows, f)(tables_flat, xcat_flat, offs)
    return out_flat.reshape(b, l, f * _EMB)
